# baseline (speedup 1.0000x reference)
import functools

import jax
import jax.numpy as jnp
from jax.experimental import pallas as pl
from jax.experimental.pallas import tpu as pltpu


def _pick_block(n, target):
    for b in range(min(target, n), 7, -1):
        if n % b == 0 and b % 8 == 0:
            return b
    return n


def _sup_body(ori_ref, aug_ref, w2_ref, out_ref):
    w2 = w2_ref[...]
    nh2 = w2.shape[1]
    out_ref[:, :nh2] = jnp.dot(ori_ref[...], w2,
                               preferred_element_type=jnp.float32)
    out_ref[:, nh2:] = jnp.dot(aug_ref[...], w2,
                               preferred_element_type=jnp.float32)


def _main_body(stg_ref, nf_ref, ns_ref, sup_ref, b2_ref, wa1_ref, ba1_ref,
               wa2_ref, wm_ref, bm_ref, wd_ref, bd_ref, gam_ref, bet_ref,
               wpi_ref, bpi_ref, wdisp_ref, bdisp_ref, wmean_ref, bmean_ref,
               emb1_ref, emb2_ref, emb_ref, pi_ref, disp_ref, mean_ref,
               h_scr, stats_ref, *, bm, chunk):
    p = pl.program_id(0)
    j = pl.program_id(1)
    n = h_scr.shape[0]
    nh1 = h_scr.shape[1]
    nh2 = b2_ref.shape[1]

    @pl.when(p == 0)
    def _encoder():
        b2 = b2_ref[...]
        sup = sup_ref[...]

        e12 = jnp.dot(stg_ref[...], sup, preferred_element_type=jnp.float32)
        e1 = e12[:, :nh2] + b2
        e2 = e12[:, nh2:] + b2
        a34 = nf_ref[...] + ns_ref[...]
        e34 = jnp.dot(a34, sup[:, :nh2],
                      preferred_element_type=jnp.float32) + 2.0 * b2

        wa1 = wa1_ref[...]
        ba1 = ba1_ref[...]
        wa2 = wa2_ref[...]

        def att(e):
            t = jnp.tanh(jnp.dot(e, wa1, preferred_element_type=jnp.float32)
                         + ba1)
            return jnp.sum(t * wa2, axis=1, keepdims=True)

        w1, w2_, w3 = att(e1), att(e34), att(e2)
        m = jnp.maximum(jnp.maximum(w1, w2_), w3)
        x1 = jnp.exp(w1 - m)
        x2 = jnp.exp(w2_ - m)
        x3 = jnp.exp(w3 - m)
        emb = (x1 * e1 + x2 * e34 + x3 * e2) / (x1 + x2 + x3)

        emb = jnp.dot(emb, wm_ref[...],
                      preferred_element_type=jnp.float32) + bm_ref[...]
        h = jnp.dot(emb, wd_ref[...],
                    preferred_element_type=jnp.float32) + bd_ref[...]

        emb1_ref[...] = e1
        emb2_ref[...] = e2
        emb_ref[...] = emb
        h_scr[pl.ds(j * bm, bm), :] = h

    @pl.when((p == 1) & (j == 0))
    def _stats():
        def sum_body(c, acc):
            hc = h_scr[pl.ds(c * chunk, chunk), :]
            return acc + jnp.sum(hc, axis=0, keepdims=True)

        s = jax.lax.fori_loop(0, n // chunk, sum_body,
                              jnp.zeros((1, nh1), jnp.float32))
        mu = s * (1.0 / n)

        def var_body(c, acc):
            hc = h_scr[pl.ds(c * chunk, chunk), :] - mu
            return acc + jnp.sum(hc * hc, axis=0, keepdims=True)

        v = jax.lax.fori_loop(0, n // chunk, var_body,
                              jnp.zeros((1, nh1), jnp.float32))
        stats_ref[0:1, :] = mu
        stats_ref[1:2, :] = jax.lax.rsqrt(v * (1.0 / n) + 1e-5)

    @pl.when(p == 1)
    def _decoder():
        mu = stats_ref[0:1, :]
        rstd = stats_ref[1:2, :]
        hb = h_scr[pl.ds(j * bm, bm), :]
        x = jnp.maximum((hb - mu) * rstd * gam_ref[...] + bet_ref[...], 0.0)

        zpi = jnp.dot(x, wpi_ref[...],
                      preferred_element_type=jnp.float32) + bpi_ref[...]
        pi_ref[...] = 1.0 / (1.0 + jnp.exp(-zpi))

        zd = jnp.dot(x, wdisp_ref[...],
                     preferred_element_type=jnp.float32) + bdisp_ref[...]
        sp = jnp.maximum(zd, 0.0) + jnp.log1p(jnp.exp(-jnp.abs(zd)))
        disp_ref[...] = jnp.clip(sp, 0.0001, 10000.0)

        zm = jnp.dot(x, wmean_ref[...],
                     preferred_element_type=jnp.float32) + bmean_ref[...]
        mean_ref[...] = jnp.clip(jnp.exp(zm), 1e-05, 1000000.0)


def _impl(stg, oridata, augdata, nfadj, nsadj, W2, b2, W_a1, b_a1, W_a2,
          W_m, b_m, W_d, b_d, bn_gamma, bn_beta, W_pi, b_pi,
          W_disp, b_disp, W_mean, b_mean, interpret=False):
    n = stg.shape[0]
    nh1 = oridata.shape[1]
    nh2 = W2.shape[1]
    na = W_a1.shape[1]
    nfeat = W_pi.shape[1]
    f32 = jnp.float32

    supcat = pl.pallas_call(
        _sup_body,
        out_shape=jax.ShapeDtypeStruct((n, 2 * nh2), f32),
        interpret=interpret,
    )(oridata, augdata, W2)

    bm = _pick_block(n, 80)
    chunk = _pick_block(n, 1000)
    g = n // bm

    def full(shape):
        return pl.BlockSpec(shape, lambda p, j: (0,) * len(shape))

    def enc_rows(width):
        return pl.BlockSpec((bm, width),
                            lambda p, j: (j * (1 - p) + (g - 1) * p, 0))

    def dec_rows(width):
        return pl.BlockSpec((bm, width), lambda p, j: (j * p, 0))

    emb1, emb2, emb, pi, disp, mean = pl.pallas_call(
        functools.partial(_main_body, bm=bm, chunk=chunk),
        grid=(2, g),
        in_specs=[
            enc_rows(n), enc_rows(n), enc_rows(n),
            full((n, 2 * nh2)), full((1, nh2)),
            full((nh2, na)), full((1, na)), full((1, na)),
            full((nh2, nh2)), full((1, nh2)),
            full((nh2, nh1)), full((1, nh1)),
            full((1, nh1)), full((1, nh1)),
            full((nh1, nfeat)), full((1, nfeat)),
            full((nh1, nfeat)), full((1, nfeat)),
            full((nh1, nfeat)), full((1, nfeat)),
        ],
        out_specs=[
            enc_rows(nh2), enc_rows(nh2), enc_rows(nh2),
            dec_rows(nfeat), dec_rows(nfeat), dec_rows(nfeat),
        ],
        out_shape=[
            jax.ShapeDtypeStruct((n, nh2), f32),
            jax.ShapeDtypeStruct((n, nh2), f32),
            jax.ShapeDtypeStruct((n, nh2), f32),
            jax.ShapeDtypeStruct((n, nfeat), f32),
            jax.ShapeDtypeStruct((n, nfeat), f32),
            jax.ShapeDtypeStruct((n, nfeat), f32),
        ],
        scratch_shapes=[
            pltpu.VMEM((n, nh1), f32),
            pltpu.VMEM((2, nh1), f32),
        ],
        compiler_params=pltpu.CompilerParams(
            vmem_limit_bytes=100 * 1024 * 1024),
        interpret=interpret,
    )(stg, nfadj, nsadj, supcat, b2.reshape(1, nh2), W_a1,
      b_a1.reshape(1, na), W_a2.reshape(1, na), W_m, b_m.reshape(1, nh2),
      W_d, b_d.reshape(1, nh1), bn_gamma.reshape(1, nh1),
      bn_beta.reshape(1, nh1), W_pi, b_pi.reshape(1, nfeat),
      W_disp, b_disp.reshape(1, nfeat), W_mean, b_mean.reshape(1, nfeat))

    return (emb1, emb2, emb, pi, disp, mean)


def kernel(stg, oridata, augdata, nfadj, nsadj, W2, b2, W_a1, b_a1, W_a2,
           W_m, b_m, W_d, b_d, bn_gamma, bn_beta, W_pi, b_pi,
           W_disp, b_disp, W_mean, b_mean):
    return _impl(stg, oridata, augdata, nfadj, nsadj, W2, b2, W_a1, b_a1,
                 W_a2, W_m, b_m, W_d, b_d, bn_gamma, bn_beta, W_pi, b_pi,
                 W_disp, b_disp, W_mean, b_mean)


# speedup vs baseline: 1.0550x; 1.0550x over previous
import functools

import jax
import jax.numpy as jnp
from jax.experimental import pallas as pl
from jax.experimental.pallas import tpu as pltpu


def _pick_block(n, target):
    for b in range(min(target, n), 7, -1):
        if n % b == 0 and b % 8 == 0:
            return b
    return n


def _sup_body(ori_ref, aug_ref, w2_ref, out_ref):
    w2 = w2_ref[...]
    nh2 = w2.shape[1]
    out_ref[:, :nh2] = jnp.dot(ori_ref[...], w2,
                               preferred_element_type=jnp.float32)
    out_ref[:, nh2:] = jnp.dot(aug_ref[...], w2,
                               preferred_element_type=jnp.float32)


def _main_body(stg_ref, nf_ref, ns_ref, sup_ref, b2_ref, wa1_ref, ba1_ref,
               wa2_ref, wm_ref, bm_ref, wd_ref, bd_ref, gam_ref, bet_ref,
               wpi_ref, bpi_ref, wdisp_ref, bdisp_ref, wmean_ref, bmean_ref,
               emb1_ref, emb2_ref, emb_ref, pi_ref, disp_ref, mean_ref,
               e_scr, stats_ref, *, bm, chunk):
    p = pl.program_id(0)
    j = pl.program_id(1)
    n = e_scr.shape[0]
    nh1 = wd_ref.shape[1]
    nh2 = b2_ref.shape[1]

    @pl.when(p == 0)
    def _encoder():
        b2 = b2_ref[...]
        sup = sup_ref[...]

        e12 = jnp.dot(stg_ref[...], sup, preferred_element_type=jnp.float32)
        e1 = e12[:, :nh2] + b2
        e2 = e12[:, nh2:] + b2
        a34 = nf_ref[...] + ns_ref[...]
        e34 = jnp.dot(a34, sup[:, :nh2],
                      preferred_element_type=jnp.float32) + 2.0 * b2

        wa1 = wa1_ref[...]
        ba1 = ba1_ref[...]
        wa2 = wa2_ref[...]

        def att(e):
            t = jnp.tanh(jnp.dot(e, wa1, preferred_element_type=jnp.float32)
                         + ba1)
            return jnp.sum(t * wa2, axis=1, keepdims=True)

        w1, w2_, w3 = att(e1), att(e34), att(e2)
        m = jnp.maximum(jnp.maximum(w1, w2_), w3)
        x1 = jnp.exp(w1 - m)
        x2 = jnp.exp(w2_ - m)
        x3 = jnp.exp(w3 - m)
        emb = (x1 * e1 + x2 * e34 + x3 * e2) / (x1 + x2 + x3)

        emb = jnp.dot(emb, wm_ref[...],
                      preferred_element_type=jnp.float32) + bm_ref[...]

        emb1_ref[...] = e1
        emb2_ref[...] = e2
        emb_ref[...] = emb
        e_scr[pl.ds(j * bm, bm), :] = emb

    def h_rows(base, width):
        return jnp.dot(e_scr[pl.ds(base, width), :], wd_ref[...],
                       preferred_element_type=jnp.float32) + bd_ref[...]

    @pl.when((p == 1) & (j == 0))
    def _stats():
        def sum_body(c, acc):
            return acc + jnp.sum(h_rows(c * chunk, chunk),
                                 axis=0, keepdims=True)

        s = jax.lax.fori_loop(0, n // chunk, sum_body,
                              jnp.zeros((1, nh1), jnp.float32))
        mu = s * (1.0 / n)

        def var_body(c, acc):
            hc = h_rows(c * chunk, chunk) - mu
            return acc + jnp.sum(hc * hc, axis=0, keepdims=True)

        v = jax.lax.fori_loop(0, n // chunk, var_body,
                              jnp.zeros((1, nh1), jnp.float32))
        stats_ref[0:1, :] = mu
        stats_ref[1:2, :] = jax.lax.rsqrt(v * (1.0 / n) + 1e-5)

    @pl.when(p == 1)
    def _decoder():
        mu = stats_ref[0:1, :]
        rstd = stats_ref[1:2, :]
        hb = h_rows(j * bm, bm)
        x = jnp.maximum((hb - mu) * rstd * gam_ref[...] + bet_ref[...], 0.0)

        zpi = jnp.dot(x, wpi_ref[...],
                      preferred_element_type=jnp.float32) + bpi_ref[...]
        pi_ref[...] = 1.0 / (1.0 + jnp.exp(-zpi))

        zd = jnp.dot(x, wdisp_ref[...],
                     preferred_element_type=jnp.float32) + bdisp_ref[...]
        sp = jnp.maximum(zd, 0.0) + jnp.log1p(jnp.exp(-jnp.abs(zd)))
        disp_ref[...] = jnp.clip(sp, 0.0001, 10000.0)

        zm = jnp.dot(x, wmean_ref[...],
                     preferred_element_type=jnp.float32) + bmean_ref[...]
        mean_ref[...] = jnp.clip(jnp.exp(zm), 1e-05, 1000000.0)


def _impl(stg, oridata, augdata, nfadj, nsadj, W2, b2, W_a1, b_a1, W_a2,
          W_m, b_m, W_d, b_d, bn_gamma, bn_beta, W_pi, b_pi,
          W_disp, b_disp, W_mean, b_mean, interpret=False):
    n = stg.shape[0]
    nh1 = oridata.shape[1]
    nh2 = W2.shape[1]
    na = W_a1.shape[1]
    nfeat = W_pi.shape[1]
    f32 = jnp.float32

    supcat = pl.pallas_call(
        _sup_body,
        out_shape=jax.ShapeDtypeStruct((n, 2 * nh2), f32),
        interpret=interpret,
    )(oridata, augdata, W2)

    bm = _pick_block(n, 200)
    chunk = _pick_block(n, 1000)
    g = n // bm

    def full(shape):
        return pl.BlockSpec(shape, lambda p, j: (0,) * len(shape))

    def enc_rows(width):
        return pl.BlockSpec((bm, width),
                            lambda p, j: (j * (1 - p) + (g - 1) * p, 0))

    def dec_rows(width):
        return pl.BlockSpec((bm, width), lambda p, j: (j * p, 0))

    emb1, emb2, emb, pi, disp, mean = pl.pallas_call(
        functools.partial(_main_body, bm=bm, chunk=chunk),
        grid=(2, g),
        in_specs=[
            enc_rows(n), enc_rows(n), enc_rows(n),
            full((n, 2 * nh2)), full((1, nh2)),
            full((nh2, na)), full((1, na)), full((1, na)),
            full((nh2, nh2)), full((1, nh2)),
            full((nh2, nh1)), full((1, nh1)),
            full((1, nh1)), full((1, nh1)),
            full((nh1, nfeat)), full((1, nfeat)),
            full((nh1, nfeat)), full((1, nfeat)),
            full((nh1, nfeat)), full((1, nfeat)),
        ],
        out_specs=[
            enc_rows(nh2), enc_rows(nh2), enc_rows(nh2),
            dec_rows(nfeat), dec_rows(nfeat), dec_rows(nfeat),
        ],
        out_shape=[
            jax.ShapeDtypeStruct((n, nh2), f32),
            jax.ShapeDtypeStruct((n, nh2), f32),
            jax.ShapeDtypeStruct((n, nh2), f32),
            jax.ShapeDtypeStruct((n, nfeat), f32),
            jax.ShapeDtypeStruct((n, nfeat), f32),
            jax.ShapeDtypeStruct((n, nfeat), f32),
        ],
        scratch_shapes=[
            pltpu.VMEM((n, nh2), f32),
            pltpu.VMEM((2, nh1), f32),
        ],
        compiler_params=pltpu.CompilerParams(
            vmem_limit_bytes=100 * 1024 * 1024),
        interpret=interpret,
    )(stg, nfadj, nsadj, supcat, b2.reshape(1, nh2), W_a1,
      b_a1.reshape(1, na), W_a2.reshape(1, na), W_m, b_m.reshape(1, nh2),
      W_d, b_d.reshape(1, nh1), bn_gamma.reshape(1, nh1),
      bn_beta.reshape(1, nh1), W_pi, b_pi.reshape(1, nfeat),
      W_disp, b_disp.reshape(1, nfeat), W_mean, b_mean.reshape(1, nfeat))

    return (emb1, emb2, emb, pi, disp, mean)


def kernel(stg, oridata, augdata, nfadj, nsadj, W2, b2, W_a1, b_a1, W_a2,
           W_m, b_m, W_d, b_d, bn_gamma, bn_beta, W_pi, b_pi,
           W_disp, b_disp, W_mean, b_mean):
    return _impl(stg, oridata, augdata, nfadj, nsadj, W2, b2, W_a1, b_a1,
                 W_a2, W_m, b_m, W_d, b_d, bn_gamma, bn_beta, W_pi, b_pi,
                 W_disp, b_disp, W_mean, b_mean)


# speedup vs baseline: 1.1041x; 1.0465x over previous
import functools

import jax
import jax.numpy as jnp
from jax.experimental import pallas as pl
from jax.experimental.pallas import tpu as pltpu


def _pick_block(n, target):
    for b in range(min(target, n), 7, -1):
        if n % b == 0 and b % 8 == 0:
            return b
    return n


def _sup_body(ori_ref, aug_ref, w2_ref, out_ref):
    w2 = w2_ref[...]
    nh2 = w2.shape[1]
    out_ref[:, :nh2] = jnp.dot(ori_ref[...], w2,
                               preferred_element_type=jnp.float32
                               ).astype(jnp.bfloat16)
    out_ref[:, nh2:] = jnp.dot(aug_ref[...], w2,
                               preferred_element_type=jnp.float32
                               ).astype(jnp.bfloat16)


def _enc_body(stg_ref, nf_ref, ns_ref, sup_ref, b2_ref, wa1_ref, ba1_ref,
              wa2_ref, wm_ref, bm_ref, wd_ref, bd_ref,
              emb1_ref, emb2_ref, emb_ref, h_ref):
    nh2 = b2_ref.shape[1]
    b2 = b2_ref[...]
    sup = sup_ref[...]

    e12 = jnp.dot(stg_ref[...].astype(jnp.bfloat16), sup,
                  preferred_element_type=jnp.float32)
    e1 = e12[:, :nh2] + b2
    e2 = e12[:, nh2:] + b2
    a34 = (nf_ref[...] + ns_ref[...]).astype(jnp.bfloat16)
    e34 = jnp.dot(a34, sup[:, :nh2],
                  preferred_element_type=jnp.float32) + 2.0 * b2

    wa1 = wa1_ref[...]
    ba1 = ba1_ref[...]
    wa2 = wa2_ref[...]

    def att(e):
        t = jnp.tanh(jnp.dot(e, wa1, preferred_element_type=jnp.float32)
                     + ba1)
        return jnp.sum(t * wa2, axis=1, keepdims=True)

    w1, w2_, w3 = att(e1), att(e34), att(e2)
    m = jnp.maximum(jnp.maximum(w1, w2_), w3)
    x1 = jnp.exp(w1 - m)
    x2 = jnp.exp(w2_ - m)
    x3 = jnp.exp(w3 - m)
    emb = (x1 * e1 + x2 * e34 + x3 * e2) / (x1 + x2 + x3)

    emb = jnp.dot(emb, wm_ref[...],
                  preferred_element_type=jnp.float32) + bm_ref[...]
    h = jnp.dot(emb, wd_ref[...],
                preferred_element_type=jnp.float32) + bd_ref[...]

    emb1_ref[...] = e1
    emb2_ref[...] = e2
    emb_ref[...] = emb
    h_ref[...] = h


def _dec_body(h_ref, gam_ref, bet_ref, wpi_ref, bpi_ref, wdisp_ref,
              bdisp_ref, wmean_ref, bmean_ref,
              pi_ref, disp_ref, mean_ref, stats_ref, *, bm2, chunk):
    i = pl.program_id(0)
    n = h_ref.shape[0]
    nh1 = h_ref.shape[1]

    @pl.when(i == 0)
    def _():
        def sum_body(j, acc):
            hc = h_ref[pl.ds(j * chunk, chunk), :]
            return acc + jnp.sum(hc, axis=0, keepdims=True)

        s = jax.lax.fori_loop(0, n // chunk, sum_body,
                              jnp.zeros((1, nh1), jnp.float32))
        mu = s * (1.0 / n)

        def var_body(j, acc):
            hc = h_ref[pl.ds(j * chunk, chunk), :] - mu
            return acc + jnp.sum(hc * hc, axis=0, keepdims=True)

        v = jax.lax.fori_loop(0, n // chunk, var_body,
                              jnp.zeros((1, nh1), jnp.float32))
        stats_ref[0:1, :] = mu
        stats_ref[1:2, :] = jax.lax.rsqrt(v * (1.0 / n) + 1e-5)

    mu = stats_ref[0:1, :]
    rstd = stats_ref[1:2, :]
    hb = h_ref[pl.ds(i * bm2, bm2), :]
    x = jnp.maximum((hb - mu) * rstd * gam_ref[...] + bet_ref[...], 0.0)

    zpi = jnp.dot(x, wpi_ref[...],
                  preferred_element_type=jnp.float32) + bpi_ref[...]
    pi_ref[...] = 1.0 / (1.0 + jnp.exp(-zpi))

    zd = jnp.dot(x, wdisp_ref[...],
                 preferred_element_type=jnp.float32) + bdisp_ref[...]
    sp = jnp.maximum(zd, 0.0) + jnp.log1p(jnp.exp(-jnp.abs(zd)))
    disp_ref[...] = jnp.clip(sp, 0.0001, 10000.0)

    zm = jnp.dot(x, wmean_ref[...],
                 preferred_element_type=jnp.float32) + bmean_ref[...]
    mean_ref[...] = jnp.clip(jnp.exp(zm), 1e-05, 1000000.0)


def _bw_body(stg_ref, nf_ref, ns_ref, out_ref):
    s = (jnp.sum(stg_ref[...], axis=1, keepdims=True)
         + jnp.sum(nf_ref[...], axis=1, keepdims=True)
         + jnp.sum(ns_ref[...], axis=1, keepdims=True))
    out_ref[...] = jnp.broadcast_to(s, out_ref.shape)


def _bw_probe(stg, nfadj, nsadj):
    n = stg.shape[0]
    bm = _pick_block(n, 200)

    def rows(width):
        return pl.BlockSpec((bm, width), lambda i: (i, 0))

    return pl.pallas_call(
        _bw_body,
        grid=(n // bm,),
        in_specs=[rows(n), rows(n), rows(n)],
        out_specs=rows(8),
        out_shape=jax.ShapeDtypeStruct((n, 8), jnp.float32),
    )(stg, nfadj, nsadj)


def _impl(stg, oridata, augdata, nfadj, nsadj, W2, b2, W_a1, b_a1, W_a2,
          W_m, b_m, W_d, b_d, bn_gamma, bn_beta, W_pi, b_pi,
          W_disp, b_disp, W_mean, b_mean, interpret=False):
    n = stg.shape[0]
    nh1 = oridata.shape[1]
    nh2 = W2.shape[1]
    na = W_a1.shape[1]
    nfeat = W_pi.shape[1]
    f32 = jnp.float32

    supcat = pl.pallas_call(
        _sup_body,
        out_shape=jax.ShapeDtypeStruct((n, 2 * nh2), jnp.bfloat16),
        interpret=interpret,
    )(oridata, augdata, W2)

    bm = _pick_block(n, 200)
    grid = n // bm

    def full(shape):
        return pl.BlockSpec(shape, lambda i: (0,) * len(shape))

    def rows(width):
        return pl.BlockSpec((bm, width), lambda i: (i, 0))

    emb1, emb2, emb, h = pl.pallas_call(
        _enc_body,
        grid=(grid,),
        in_specs=[
            rows(n), rows(n), rows(n),
            full((n, 2 * nh2)), full((1, nh2)),
            full((nh2, na)), full((1, na)), full((1, na)),
            full((nh2, nh2)), full((1, nh2)),
            full((nh2, nh1)), full((1, nh1)),
        ],
        out_specs=[rows(nh2), rows(nh2), rows(nh2), rows(nh1)],
        out_shape=[
            jax.ShapeDtypeStruct((n, nh2), f32),
            jax.ShapeDtypeStruct((n, nh2), f32),
            jax.ShapeDtypeStruct((n, nh2), f32),
            jax.ShapeDtypeStruct((n, nh1), f32),
        ],
        interpret=interpret,
    )(stg, nfadj, nsadj, supcat, b2.reshape(1, nh2), W_a1,
      b_a1.reshape(1, na), W_a2.reshape(1, na), W_m, b_m.reshape(1, nh2),
      W_d, b_d.reshape(1, nh1))

    bm2 = _pick_block(n, 1000)
    chunk = _pick_block(n, 1000)
    grid2 = n // bm2

    def rows2(width):
        return pl.BlockSpec((bm2, width), lambda i: (i, 0))

    pi, disp, mean = pl.pallas_call(
        functools.partial(_dec_body, bm2=bm2, chunk=chunk),
        grid=(grid2,),
        in_specs=[
            full((n, nh1)), full((1, nh1)), full((1, nh1)),
            full((nh1, nfeat)), full((1, nfeat)),
            full((nh1, nfeat)), full((1, nfeat)),
            full((nh1, nfeat)), full((1, nfeat)),
        ],
        out_specs=[rows2(nfeat), rows2(nfeat), rows2(nfeat)],
        out_shape=[
            jax.ShapeDtypeStruct((n, nfeat), f32),
            jax.ShapeDtypeStruct((n, nfeat), f32),
            jax.ShapeDtypeStruct((n, nfeat), f32),
        ],
        scratch_shapes=[pltpu.VMEM((2, nh1), f32)],
        interpret=interpret,
    )(h, bn_gamma.reshape(1, nh1), bn_beta.reshape(1, nh1),
      W_pi, b_pi.reshape(1, nfeat), W_disp, b_disp.reshape(1, nfeat),
      W_mean, b_mean.reshape(1, nfeat))

    return (emb1, emb2, emb, pi, disp, mean)


def kernel(stg, oridata, augdata, nfadj, nsadj, W2, b2, W_a1, b_a1, W_a2,
           W_m, b_m, W_d, b_d, bn_gamma, bn_beta, W_pi, b_pi,
           W_disp, b_disp, W_mean, b_mean):
    return _impl(stg, oridata, augdata, nfadj, nsadj, W2, b2, W_a1, b_a1,
                 W_a2, W_m, b_m, W_d, b_d, bn_gamma, bn_beta, W_pi, b_pi,
                 W_disp, b_disp, W_mean, b_mean)


# speedup vs baseline: 1.1044x; 1.0003x over previous
import functools

import jax
import jax.numpy as jnp
from jax.experimental import pallas as pl
from jax.experimental.pallas import tpu as pltpu


def _pick_block(n, target):
    for b in range(min(target, n), 7, -1):
        if n % b == 0 and b % 8 == 0:
            return b
    return n


def _sup_body(ori_ref, aug_ref, w2_ref, out_ref):
    w2 = w2_ref[...]
    nh2 = w2.shape[1]
    out_ref[:, :nh2] = jnp.dot(ori_ref[...], w2,
                               preferred_element_type=jnp.float32
                               ).astype(jnp.bfloat16)
    out_ref[:, nh2:] = jnp.dot(aug_ref[...], w2,
                               preferred_element_type=jnp.float32
                               ).astype(jnp.bfloat16)


def _enc_body(stg_ref, nf_ref, ns_ref, sup_ref, b2_ref, wa1_ref, ba1_ref,
              wa2_ref, wm_ref, bm_ref, wd_ref, bd_ref,
              emb1_ref, emb2_ref, emb_ref, h_ref):
    nh2 = b2_ref.shape[1]
    b2 = b2_ref[...]
    sup = sup_ref[...]

    e12 = jnp.dot(stg_ref[...].astype(jnp.bfloat16), sup,
                  preferred_element_type=jnp.float32)
    e1 = e12[:, :nh2] + b2
    e2 = e12[:, nh2:] + b2
    a34 = (nf_ref[...] + ns_ref[...]).astype(jnp.bfloat16)
    e34 = jnp.dot(a34, sup[:, :nh2],
                  preferred_element_type=jnp.float32) + 2.0 * b2

    wa1 = wa1_ref[...]
    ba1 = ba1_ref[...]
    wa2 = wa2_ref[...]

    def att(e):
        t = jnp.tanh(jnp.dot(e, wa1, preferred_element_type=jnp.float32)
                     + ba1)
        return jnp.sum(t * wa2, axis=1, keepdims=True)

    w1, w2_, w3 = att(e1), att(e34), att(e2)
    m = jnp.maximum(jnp.maximum(w1, w2_), w3)
    x1 = jnp.exp(w1 - m)
    x2 = jnp.exp(w2_ - m)
    x3 = jnp.exp(w3 - m)
    emb = (x1 * e1 + x2 * e34 + x3 * e2) / (x1 + x2 + x3)

    emb = jnp.dot(emb, wm_ref[...],
                  preferred_element_type=jnp.float32) + bm_ref[...]
    h = jnp.dot(emb, wd_ref[...],
                preferred_element_type=jnp.float32) + bd_ref[...]

    emb1_ref[...] = e1
    emb2_ref[...] = e2
    emb_ref[...] = emb
    h_ref[...] = h


def _dec_body(h_ref, gam_ref, bet_ref, wpi_ref, bpi_ref, wdisp_ref,
              bdisp_ref, wmean_ref, bmean_ref,
              pi_ref, disp_ref, mean_ref, stats_ref, *, bm2, chunk):
    i = pl.program_id(0)
    n = h_ref.shape[0]
    nh1 = h_ref.shape[1]

    @pl.when(i == 0)
    def _():
        def sum_body(j, acc):
            hc = h_ref[pl.ds(j * chunk, chunk), :]
            return acc + jnp.sum(hc, axis=0, keepdims=True)

        s = jax.lax.fori_loop(0, n // chunk, sum_body,
                              jnp.zeros((1, nh1), jnp.float32))
        mu = s * (1.0 / n)

        def var_body(j, acc):
            hc = h_ref[pl.ds(j * chunk, chunk), :] - mu
            return acc + jnp.sum(hc * hc, axis=0, keepdims=True)

        v = jax.lax.fori_loop(0, n // chunk, var_body,
                              jnp.zeros((1, nh1), jnp.float32))
        stats_ref[0:1, :] = mu
        stats_ref[1:2, :] = jax.lax.rsqrt(v * (1.0 / n) + 1e-5)

    mu = stats_ref[0:1, :]
    rstd = stats_ref[1:2, :]
    hb = h_ref[pl.ds(i * bm2, bm2), :]
    x = jnp.maximum((hb - mu) * rstd * gam_ref[...] + bet_ref[...], 0.0)

    zpi = jnp.dot(x, wpi_ref[...],
                  preferred_element_type=jnp.float32) + bpi_ref[...]
    pi_ref[...] = 1.0 / (1.0 + jnp.exp(-zpi))

    zd = jnp.dot(x, wdisp_ref[...],
                 preferred_element_type=jnp.float32) + bdisp_ref[...]
    sp = jnp.maximum(zd, 0.0) + jnp.log1p(jnp.exp(-jnp.abs(zd)))
    disp_ref[...] = jnp.clip(sp, 0.0001, 10000.0)

    zm = jnp.dot(x, wmean_ref[...],
                 preferred_element_type=jnp.float32) + bmean_ref[...]
    mean_ref[...] = jnp.clip(jnp.exp(zm), 1e-05, 1000000.0)


def _impl(stg, oridata, augdata, nfadj, nsadj, W2, b2, W_a1, b_a1, W_a2,
          W_m, b_m, W_d, b_d, bn_gamma, bn_beta, W_pi, b_pi,
          W_disp, b_disp, W_mean, b_mean, interpret=False):
    n = stg.shape[0]
    nh1 = oridata.shape[1]
    nh2 = W2.shape[1]
    na = W_a1.shape[1]
    nfeat = W_pi.shape[1]
    f32 = jnp.float32

    bs = _pick_block(n, 2504)
    supcat = pl.pallas_call(
        _sup_body,
        grid=(n // bs,),
        in_specs=[
            pl.BlockSpec((bs, nh1), lambda i: (i, 0)),
            pl.BlockSpec((bs, nh1), lambda i: (i, 0)),
            pl.BlockSpec((nh1, nh2), lambda i: (0, 0)),
        ],
        out_specs=pl.BlockSpec((bs, 2 * nh2), lambda i: (i, 0)),
        out_shape=jax.ShapeDtypeStruct((n, 2 * nh2), jnp.bfloat16),
        interpret=interpret,
    )(oridata, augdata, W2)

    bm = _pick_block(n, 200)
    grid = n // bm

    def full(shape):
        return pl.BlockSpec(shape, lambda i: (0,) * len(shape))

    def rows(width):
        return pl.BlockSpec((bm, width), lambda i: (i, 0))

    emb1, emb2, emb, h = pl.pallas_call(
        _enc_body,
        grid=(grid,),
        in_specs=[
            rows(n), rows(n), rows(n),
            full((n, 2 * nh2)), full((1, nh2)),
            full((nh2, na)), full((1, na)), full((1, na)),
            full((nh2, nh2)), full((1, nh2)),
            full((nh2, nh1)), full((1, nh1)),
        ],
        out_specs=[rows(nh2), rows(nh2), rows(nh2), rows(nh1)],
        out_shape=[
            jax.ShapeDtypeStruct((n, nh2), f32),
            jax.ShapeDtypeStruct((n, nh2), f32),
            jax.ShapeDtypeStruct((n, nh2), f32),
            jax.ShapeDtypeStruct((n, nh1), f32),
        ],
        interpret=interpret,
    )(stg, nfadj, nsadj, supcat, b2.reshape(1, nh2), W_a1,
      b_a1.reshape(1, na), W_a2.reshape(1, na), W_m, b_m.reshape(1, nh2),
      W_d, b_d.reshape(1, nh1))

    bm2 = _pick_block(n, 1000)
    chunk = _pick_block(n, 1000)
    grid2 = n // bm2

    def rows2(width):
        return pl.BlockSpec((bm2, width), lambda i: (i, 0))

    pi, disp, mean = pl.pallas_call(
        functools.partial(_dec_body, bm2=bm2, chunk=chunk),
        grid=(grid2,),
        in_specs=[
            full((n, nh1)), full((1, nh1)), full((1, nh1)),
            full((nh1, nfeat)), full((1, nfeat)),
            full((nh1, nfeat)), full((1, nfeat)),
            full((nh1, nfeat)), full((1, nfeat)),
        ],
        out_specs=[rows2(nfeat), rows2(nfeat), rows2(nfeat)],
        out_shape=[
            jax.ShapeDtypeStruct((n, nfeat), f32),
            jax.ShapeDtypeStruct((n, nfeat), f32),
            jax.ShapeDtypeStruct((n, nfeat), f32),
        ],
        scratch_shapes=[pltpu.VMEM((2, nh1), f32)],
        interpret=interpret,
    )(h, bn_gamma.reshape(1, nh1), bn_beta.reshape(1, nh1),
      W_pi, b_pi.reshape(1, nfeat), W_disp, b_disp.reshape(1, nfeat),
      W_mean, b_mean.reshape(1, nfeat))

    return (emb1, emb2, emb, pi, disp, mean)


def kernel(stg, oridata, augdata, nfadj, nsadj, W2, b2, W_a1, b_a1, W_a2,
           W_m, b_m, W_d, b_d, bn_gamma, bn_beta, W_pi, b_pi,
           W_disp, b_disp, W_mean, b_mean):
    return _impl(stg, oridata, augdata, nfadj, nsadj, W2, b2, W_a1, b_a1,
                 W_a2, W_m, b_m, W_d, b_d, bn_gamma, bn_beta, W_pi, b_pi,
                 W_disp, b_disp, W_mean, b_mean)


# speedup vs baseline: 1.1118x; 1.0067x over previous
import functools

import jax
import jax.numpy as jnp
from jax.experimental import pallas as pl
from jax.experimental.pallas import tpu as pltpu


def _pick_block(n, target):
    for b in range(min(target, n), 7, -1):
        if n % b == 0 and b % 8 == 0:
            return b
    return n


def _sup_body(ori_ref, aug_ref, w2_ref, out_ref):
    w2 = w2_ref[...]
    nh2 = w2.shape[1]
    out_ref[:, :nh2] = jnp.dot(ori_ref[...], w2,
                               preferred_element_type=jnp.float32
                               ).astype(jnp.bfloat16)
    out_ref[:, nh2:] = jnp.dot(aug_ref[...], w2,
                               preferred_element_type=jnp.float32
                               ).astype(jnp.bfloat16)


def _enc_body(stg_ref, nf_ref, ns_ref, sup_ref, b2_ref, wa1_ref, ba1_ref,
              wa2_ref, wm_ref, bm_ref, wd_ref, bd_ref,
              emb1_ref, emb2_ref, emb_ref, h_ref):
    nh2 = b2_ref.shape[1]
    b2 = b2_ref[...]
    sup = sup_ref[...]

    e12 = jnp.dot(stg_ref[...].astype(jnp.bfloat16), sup,
                  preferred_element_type=jnp.float32)
    e1 = e12[:, :nh2] + b2
    e2 = e12[:, nh2:] + b2
    a34 = (nf_ref[...] + ns_ref[...]).astype(jnp.bfloat16)
    e34 = jnp.dot(a34, sup[:, :nh2],
                  preferred_element_type=jnp.float32) + 2.0 * b2

    wa1 = wa1_ref[...]
    ba1 = ba1_ref[...]
    wa2 = wa2_ref[...]

    def att(e):
        t = jnp.tanh(jnp.dot(e, wa1, preferred_element_type=jnp.float32)
                     + ba1)
        return jnp.sum(t * wa2, axis=1, keepdims=True)

    w1, w2_, w3 = att(e1), att(e34), att(e2)
    m = jnp.maximum(jnp.maximum(w1, w2_), w3)
    x1 = jnp.exp(w1 - m)
    x2 = jnp.exp(w2_ - m)
    x3 = jnp.exp(w3 - m)
    emb = (x1 * e1 + x2 * e34 + x3 * e2) / (x1 + x2 + x3)

    emb = jnp.dot(emb, wm_ref[...],
                  preferred_element_type=jnp.float32) + bm_ref[...]
    h = jnp.dot(emb, wd_ref[...],
                preferred_element_type=jnp.float32) + bd_ref[...]

    emb1_ref[...] = e1
    emb2_ref[...] = e2
    emb_ref[...] = emb
    h_ref[...] = h.astype(jnp.bfloat16)


def _dec_body(h_ref, gam_ref, bet_ref, wpi_ref, bpi_ref, wdisp_ref,
              bdisp_ref, wmean_ref, bmean_ref,
              pi_ref, disp_ref, mean_ref, stats_ref, *, bm2, chunk):
    i = pl.program_id(0)
    n = h_ref.shape[0]
    nh1 = h_ref.shape[1]

    @pl.when(i == 0)
    def _():
        def sum_body(j, acc):
            hc = h_ref[pl.ds(j * chunk, chunk), :].astype(jnp.float32)
            return acc + jnp.sum(hc, axis=0, keepdims=True)

        s = jax.lax.fori_loop(0, n // chunk, sum_body,
                              jnp.zeros((1, nh1), jnp.float32))
        mu = s * (1.0 / n)

        def var_body(j, acc):
            hc = h_ref[pl.ds(j * chunk, chunk), :].astype(jnp.float32) - mu
            return acc + jnp.sum(hc * hc, axis=0, keepdims=True)

        v = jax.lax.fori_loop(0, n // chunk, var_body,
                              jnp.zeros((1, nh1), jnp.float32))
        stats_ref[0:1, :] = mu
        stats_ref[1:2, :] = jax.lax.rsqrt(v * (1.0 / n) + 1e-5)

    mu = stats_ref[0:1, :]
    rstd = stats_ref[1:2, :]
    hb = h_ref[pl.ds(i * bm2, bm2), :].astype(jnp.float32)
    x = jnp.maximum((hb - mu) * rstd * gam_ref[...] + bet_ref[...], 0.0)

    zpi = jnp.dot(x, wpi_ref[...],
                  preferred_element_type=jnp.float32) + bpi_ref[...]
    pi_ref[...] = 1.0 / (1.0 + jnp.exp(-zpi))

    zd = jnp.dot(x, wdisp_ref[...],
                 preferred_element_type=jnp.float32) + bdisp_ref[...]
    sp = jnp.maximum(zd, 0.0) + jnp.log1p(jnp.exp(-jnp.abs(zd)))
    disp_ref[...] = jnp.clip(sp, 0.0001, 10000.0)

    zm = jnp.dot(x, wmean_ref[...],
                 preferred_element_type=jnp.float32) + bmean_ref[...]
    mean_ref[...] = jnp.clip(jnp.exp(zm), 1e-05, 1000000.0)


def _impl(stg, oridata, augdata, nfadj, nsadj, W2, b2, W_a1, b_a1, W_a2,
          W_m, b_m, W_d, b_d, bn_gamma, bn_beta, W_pi, b_pi,
          W_disp, b_disp, W_mean, b_mean, interpret=False):
    n = stg.shape[0]
    nh1 = oridata.shape[1]
    nh2 = W2.shape[1]
    na = W_a1.shape[1]
    nfeat = W_pi.shape[1]
    f32 = jnp.float32

    bs = _pick_block(n, 2504)
    supcat = pl.pallas_call(
        _sup_body,
        grid=(n // bs,),
        in_specs=[
            pl.BlockSpec((bs, nh1), lambda i: (i, 0)),
            pl.BlockSpec((bs, nh1), lambda i: (i, 0)),
            pl.BlockSpec((nh1, nh2), lambda i: (0, 0)),
        ],
        out_specs=pl.BlockSpec((bs, 2 * nh2), lambda i: (i, 0)),
        out_shape=jax.ShapeDtypeStruct((n, 2 * nh2), jnp.bfloat16),
        interpret=interpret,
    )(oridata, augdata, W2)

    bm = _pick_block(n, 200)
    grid = n // bm

    def full(shape):
        return pl.BlockSpec(shape, lambda i: (0,) * len(shape))

    def rows(width):
        return pl.BlockSpec((bm, width), lambda i: (i, 0))

    emb1, emb2, emb, h = pl.pallas_call(
        _enc_body,
        grid=(grid,),
        in_specs=[
            rows(n), rows(n), rows(n),
            full((n, 2 * nh2)), full((1, nh2)),
            full((nh2, na)), full((1, na)), full((1, na)),
            full((nh2, nh2)), full((1, nh2)),
            full((nh2, nh1)), full((1, nh1)),
        ],
        out_specs=[rows(nh2), rows(nh2), rows(nh2), rows(nh1)],
        out_shape=[
            jax.ShapeDtypeStruct((n, nh2), f32),
            jax.ShapeDtypeStruct((n, nh2), f32),
            jax.ShapeDtypeStruct((n, nh2), f32),
            jax.ShapeDtypeStruct((n, nh1), jnp.bfloat16),
        ],
        interpret=interpret,
    )(stg, nfadj, nsadj, supcat, b2.reshape(1, nh2), W_a1,
      b_a1.reshape(1, na), W_a2.reshape(1, na), W_m, b_m.reshape(1, nh2),
      W_d, b_d.reshape(1, nh1))

    bm2 = _pick_block(n, 1000)
    chunk = _pick_block(n, 1000)
    grid2 = n // bm2

    def rows2(width):
        return pl.BlockSpec((bm2, width), lambda i: (i, 0))

    pi, disp, mean = pl.pallas_call(
        functools.partial(_dec_body, bm2=bm2, chunk=chunk),
        grid=(grid2,),
        in_specs=[
            full((n, nh1)), full((1, nh1)), full((1, nh1)),
            full((nh1, nfeat)), full((1, nfeat)),
            full((nh1, nfeat)), full((1, nfeat)),
            full((nh1, nfeat)), full((1, nfeat)),
        ],
        out_specs=[rows2(nfeat), rows2(nfeat), rows2(nfeat)],
        out_shape=[
            jax.ShapeDtypeStruct((n, nfeat), f32),
            jax.ShapeDtypeStruct((n, nfeat), f32),
            jax.ShapeDtypeStruct((n, nfeat), f32),
        ],
        scratch_shapes=[pltpu.VMEM((2, nh1), f32)],
        interpret=interpret,
    )(h, bn_gamma.reshape(1, nh1), bn_beta.reshape(1, nh1),
      W_pi, b_pi.reshape(1, nfeat), W_disp, b_disp.reshape(1, nfeat),
      W_mean, b_mean.reshape(1, nfeat))

    return (emb1, emb2, emb, pi, disp, mean)


def kernel(stg, oridata, augdata, nfadj, nsadj, W2, b2, W_a1, b_a1, W_a2,
           W_m, b_m, W_d, b_d, bn_gamma, bn_beta, W_pi, b_pi,
           W_disp, b_disp, W_mean, b_mean):
    return _impl(stg, oridata, augdata, nfadj, nsadj, W2, b2, W_a1, b_a1,
                 W_a2, W_m, b_m, W_d, b_d, bn_gamma, bn_beta, W_pi, b_pi,
                 W_disp, b_disp, W_mean, b_mean)
